# baseline (speedup 1.0000x reference)
import functools

import jax
import jax.numpy as jnp
from jax import lax
from jax.experimental import pallas as pl
from jax.experimental.pallas import tpu as pltpu
from jax.experimental.pallas import tpu_sc as plsc

D = 1024
SCALE = 32.0

NC, NS, L = 2, 16, 16
NW = NC * NS

B = 16384
B_PER_W = B // NW
CB = 32
NCHUNK = B_PER_W // CB
SLICES_PER_ROW = D // L
NBUF = 3


def _scale_chunk(rows_v):
    def row_body(r, c0):
        for c in range(SLICES_PER_ROW):
            sl = pl.ds(c * L, L)
            rows_v[r, sl] = rows_v[r, sl] * SCALE
        return c0

    lax.fori_loop(0, CB, row_body, 0)


def _sc_embed(idx_hbm, table_hbm, out_hbm, idx_v, rows0, rows1, rows2,
              sg0, sg1, sg2, ss0, ss1, ss2):
    wid = lax.axis_index("s") * NC + lax.axis_index("c")
    base = wid * B_PER_W
    pltpu.sync_copy(idx_hbm.at[pl.ds(base, B_PER_W)], idx_v)

    bufs = (rows0, rows1, rows2)
    gsems = (sg0, sg1, sg2)
    ssems = (ss0, ss1, ss2)

    def gather_desc(g):
        b = g % NBUF
        return pltpu.make_async_copy(
            table_hbm.at[idx_v.at[pl.ds(g * CB, CB)]], bufs[b], gsems[b]
        )

    def scatter_desc(g):
        b = g % NBUF
        return pltpu.make_async_copy(
            bufs[b], out_hbm.at[pl.ds(base + g * CB, CB)], ssems[b]
        )

    for g in range(NBUF):
        gather_desc(g).start()

    for g in range(NCHUNK):
        gather_desc(g).wait()
        _scale_chunk(bufs[g % NBUF])
        scatter_desc(g).start()
        nxt = g + NBUF
        if nxt < NCHUNK:
            scatter_desc(nxt - NBUF).wait()
            gather_desc(nxt).start()

    for g in range(NCHUNK - NBUF, NCHUNK):
        scatter_desc(g).wait()


@functools.partial(
    pl.kernel,
    mesh=plsc.VectorSubcoreMesh(core_axis_name="c", subcore_axis_name="s"),
    out_type=jax.ShapeDtypeStruct((B, D), jnp.float32),
    scratch_types=[
        pltpu.VMEM((B_PER_W,), jnp.int32),
        pltpu.VMEM((CB, D), jnp.float32),
        pltpu.VMEM((CB, D), jnp.float32),
        pltpu.VMEM((CB, D), jnp.float32),
        pltpu.SemaphoreType.DMA,
        pltpu.SemaphoreType.DMA,
        pltpu.SemaphoreType.DMA,
        pltpu.SemaphoreType.DMA,
        pltpu.SemaphoreType.DMA,
        pltpu.SemaphoreType.DMA,
    ],
)
def _embed_call(idx_hbm, table_hbm, out_hbm, idx_v, rows0, rows1, rows2,
                sg0, sg1, sg2, ss0, ss1, ss2):
    _sc_embed(idx_hbm, table_hbm, out_hbm, idx_v, rows0, rows1, rows2,
              sg0, sg1, sg2, ss0, ss1, ss2)


def kernel(input_ids, weight):
    idx = input_ids.reshape(-1).astype(jnp.int32)
    out = _embed_call(idx, weight)
    return out.reshape(input_ids.shape + (D,))


# speedup vs baseline: 1.0349x; 1.0349x over previous
import functools

import jax
import jax.numpy as jnp
from jax import lax
from jax.experimental import pallas as pl
from jax.experimental.pallas import tpu as pltpu
from jax.experimental.pallas import tpu_sc as plsc

D = 1024
SCALE = 32.0

NC, NS, L = 2, 16, 16
NW = NC * NS

B = 16384
B_PER_W = B // NW
CB = 16
NCHUNK = B_PER_W // CB
SLICES_PER_ROW = D // L
NBUF = 4


def _scale_chunk(rows_v):
    def row_body(r, c0):
        for c in range(SLICES_PER_ROW):
            sl = pl.ds(c * L, L)
            rows_v[r, sl] = rows_v[r, sl] * SCALE
        return c0

    lax.fori_loop(0, CB, row_body, 0)


def _sc_embed(idx_hbm, table_hbm, out_hbm, idx_v, rows0, rows1, rows2,
              rows3, sg0, sg1, sg2, sg3, ss0, ss1, ss2, ss3):
    wid = lax.axis_index("s") * NC + lax.axis_index("c")
    base = wid * B_PER_W
    pltpu.sync_copy(idx_hbm.at[pl.ds(base, B_PER_W)], idx_v)

    bufs = (rows0, rows1, rows2, rows3)
    gsems = (sg0, sg1, sg2, sg3)
    ssems = (ss0, ss1, ss2, ss3)

    def gather_desc(g, b):
        return pltpu.make_async_copy(
            table_hbm.at[idx_v.at[pl.ds(g * CB, CB)]], bufs[b], gsems[b]
        )

    def scatter_desc(g, b):
        return pltpu.make_async_copy(
            bufs[b], out_hbm.at[pl.ds(base + g * CB, CB)], ssems[b]
        )

    for g in range(NBUF - 1):
        gather_desc(g, g).start()

    def ring_body(g0, carry):
        for bsel in range(NBUF):
            g = g0 * NBUF + bsel
            gather_desc(g, bsel).wait()

            prev = (bsel - 1) % NBUF

            @pl.when(g >= 1)
            def _():
                scatter_desc(g - 1, prev).wait()

            @pl.when(g + NBUF - 1 < NCHUNK)
            def _():
                gather_desc(g + NBUF - 1, prev).start()

            _scale_chunk(bufs[bsel])
            scatter_desc(g, bsel).start()
        return carry

    lax.fori_loop(0, NCHUNK // NBUF, ring_body, 0)
    scatter_desc(NCHUNK - 1, (NCHUNK - 1) % NBUF).wait()


@functools.partial(
    pl.kernel,
    mesh=plsc.VectorSubcoreMesh(core_axis_name="c", subcore_axis_name="s"),
    out_type=jax.ShapeDtypeStruct((B, D), jnp.float32),
    scratch_types=[
        pltpu.VMEM((B_PER_W,), jnp.int32),
        pltpu.VMEM((CB, D), jnp.float32),
        pltpu.VMEM((CB, D), jnp.float32),
        pltpu.VMEM((CB, D), jnp.float32),
        pltpu.VMEM((CB, D), jnp.float32),
        pltpu.SemaphoreType.DMA,
        pltpu.SemaphoreType.DMA,
        pltpu.SemaphoreType.DMA,
        pltpu.SemaphoreType.DMA,
        pltpu.SemaphoreType.DMA,
        pltpu.SemaphoreType.DMA,
        pltpu.SemaphoreType.DMA,
        pltpu.SemaphoreType.DMA,
    ],
)
def _embed_call(idx_hbm, table_hbm, out_hbm, idx_v, rows0, rows1, rows2,
                rows3, sg0, sg1, sg2, sg3, ss0, ss1, ss2, ss3):
    _sc_embed(idx_hbm, table_hbm, out_hbm, idx_v, rows0, rows1, rows2,
              rows3, sg0, sg1, sg2, sg3, ss0, ss1, ss2, ss3)


def kernel(input_ids, weight):
    idx = input_ids.reshape(-1).astype(jnp.int32)
    out = _embed_call(idx, weight)
    return out.reshape(input_ids.shape + (D,))


# speedup vs baseline: 1.0708x; 1.0347x over previous
import functools

import jax
import jax.numpy as jnp
from jax import lax
from jax.experimental import pallas as pl
from jax.experimental.pallas import tpu as pltpu
from jax.experimental.pallas import tpu_sc as plsc

D = 1024
SCALE = 32.0

NC, NS, L = 2, 16, 16
NW = NC * NS

B = 16384
B_PER_W = B // NW
CB = 32
NCHUNK = B_PER_W // CB
SLICES_PER_ROW = D // L


def _scale_chunk(rows_v):
    def row_body(r, c0):
        for c in range(SLICES_PER_ROW):
            sl = pl.ds(c * L, L)
            rows_v[r, sl] = rows_v[r, sl] * SCALE
        return c0

    lax.fori_loop(0, CB, row_body, 0)


def _sc_embed(idx_hbm, table_hbm, out_hbm, idx_v, rows0, rows1, sg0, sg1,
              ss0, ss1):
    wid = lax.axis_index("s") * NC + lax.axis_index("c")
    base = wid * B_PER_W
    pltpu.sync_copy(idx_hbm.at[pl.ds(base, B_PER_W)], idx_v)

    bufs = (rows0, rows1)
    gsems = (sg0, sg1)
    ssems = (ss0, ss1)

    def gather_desc(g, b):
        return pltpu.make_async_copy(
            table_hbm.at[idx_v.at[pl.ds(g * CB, CB)]], bufs[b], gsems[b]
        )

    def scatter_desc(g, b):
        return pltpu.make_async_copy(
            bufs[b], out_hbm.at[pl.ds(base + g * CB, CB)], ssems[b]
        )

    gather_desc(0, 0).start()

    def chunk_pair(g0, carry):
        for bsel in range(2):
            g = g0 * 2 + bsel
            gather_desc(g, bsel).wait()
            @pl.when(g >= 1)
            def _():
                scatter_desc(g - 1, 1 - bsel).wait()

            @pl.when(g + 1 < NCHUNK)
            def _():
                gather_desc(g + 1, 1 - bsel).start()

            _scale_chunk(bufs[bsel])
            scatter_desc(g, bsel).start()
        return carry

    lax.fori_loop(0, NCHUNK // 2, chunk_pair, 0)
    scatter_desc(NCHUNK - 1, 1).wait()


@functools.partial(
    pl.kernel,
    mesh=plsc.VectorSubcoreMesh(core_axis_name="c", subcore_axis_name="s"),
    out_type=jax.ShapeDtypeStruct((B, D), jnp.float32),
    scratch_types=[
        pltpu.VMEM((B_PER_W,), jnp.int32),
        pltpu.VMEM((CB, D), jnp.float32),
        pltpu.VMEM((CB, D), jnp.float32),
        pltpu.SemaphoreType.DMA,
        pltpu.SemaphoreType.DMA,
        pltpu.SemaphoreType.DMA,
        pltpu.SemaphoreType.DMA,
    ],
)
def _embed_call(idx_hbm, table_hbm, out_hbm, idx_v, rows0, rows1, sg0, sg1,
                ss0, ss1):
    _sc_embed(idx_hbm, table_hbm, out_hbm, idx_v, rows0, rows1, sg0, sg1,
              ss0, ss1)


def kernel(input_ids, weight):
    idx = input_ids.reshape(-1).astype(jnp.int32)
    out = _embed_call(idx, weight)
    return out.reshape(input_ids.shape + (D,))
